# baseline (speedup 1.0000x reference)
import functools

import jax
import jax.numpy as jnp
from jax import lax
from jax.experimental import pallas as pl
from jax.experimental.pallas import tpu as pltpu
from jax.experimental.pallas import tpu_sc as plsc

ROWS = 128
COLS = 32768
NC = 2
NS = 16
L = 16
NW = NC * NS
RPW = ROWS // NW
STEPS = COLS // L

_mesh = plsc.VectorSubcoreMesh(core_axis_name="c", subcore_axis_name="s")


@functools.partial(
    pl.kernel,
    mesh=_mesh,
    out_type=jax.ShapeDtypeStruct((NW, L), jnp.int32),
    scratch_types=[
        pltpu.VMEM((2, COLS), jnp.float32),
        pltpu.VMEM((L,), jnp.int32),
        pltpu.SemaphoreType.DMA,
        pltpu.SemaphoreType.DMA,
    ],
)
def _argmax_sc(x_hbm, out_hbm, buf, res, sem0, sem1):
    wid = lax.axis_index("s") * NC + lax.axis_index("c")
    base = wid * RPW
    sems = (sem0, sem1)

    copies = [pltpu.async_copy(x_hbm.at[base], buf.at[0], sems[0])]
    iota = lax.iota(jnp.int32, L)
    ansvec = jnp.zeros((L,), jnp.int32)

    for r in range(RPW):
        if r + 1 < RPW:
            copies.append(
                pltpu.async_copy(
                    x_hbm.at[base + (r + 1)], buf.at[(r + 1) % 2], sems[(r + 1) % 2]
                )
            )
        copies[r].wait()
        row = buf.at[r % 2]

        def body(j, carry):
            vmax, vidx, vcur = carry
            v = row[pl.ds(j * L, L)]
            m = v > vmax
            vmax = jnp.where(m, v, vmax)
            vidx = jnp.where(m, vcur, vidx)
            vcur = vcur + L
            return vmax, vidx, vcur

        init = (
            jnp.full((L,), -jnp.inf, jnp.float32),
            jnp.zeros((L,), jnp.int32),
            iota,
        )
        vmax, vidx, _ = lax.fori_loop(0, STEPS, body, init)

        gmax = vmax
        for shift in (1, 2, 4, 8):
            perm = iota ^ shift
            gmax = jnp.maximum(gmax, gmax.at[perm].get(mode="promise_in_bounds"))
        cand = jnp.where(vmax == gmax, vidx, COLS)
        for shift in (1, 2, 4, 8):
            perm = iota ^ shift
            cand = jnp.minimum(cand, cand.at[perm].get(mode="promise_in_bounds"))
        ansvec = jnp.where(iota == r, cand, ansvec)

    res[...] = ansvec
    pltpu.sync_copy(res, out_hbm.at[wid])


def kernel(x):
    out2d = _argmax_sc(x)
    return out2d[:, :RPW].reshape(ROWS)


# speedup vs baseline: 1.5047x; 1.5047x over previous
import functools

import jax
import jax.numpy as jnp
from jax import lax
from jax.experimental import pallas as pl
from jax.experimental.pallas import tpu as pltpu
from jax.experimental.pallas import tpu_sc as plsc

ROWS = 128
COLS = 32768
NC = 2
NS = 16
L = 16
NW = NC * NS
RPW = ROWS // NW
STEPS = COLS // L

_mesh = plsc.VectorSubcoreMesh(core_axis_name="c", subcore_axis_name="s")


@functools.partial(
    pl.kernel,
    mesh=_mesh,
    out_type=jax.ShapeDtypeStruct((NW, L), jnp.int32),
    scratch_types=[
        pltpu.VMEM((2, COLS), jnp.float32),
        pltpu.VMEM((L,), jnp.int32),
        pltpu.SemaphoreType.DMA,
        pltpu.SemaphoreType.DMA,
    ],
)
def _argmax_sc(x_hbm, out_hbm, buf, res, sem0, sem1):
    wid = lax.axis_index("s") * NC + lax.axis_index("c")
    base = wid * RPW
    sems = (sem0, sem1)

    copies = [pltpu.async_copy(x_hbm.at[base], buf.at[0], sems[0])]
    iota = lax.iota(jnp.int32, L)
    ansvec = jnp.zeros((L,), jnp.int32)

    for r in range(RPW):
        if r + 1 < RPW:
            copies.append(
                pltpu.async_copy(
                    x_hbm.at[base + (r + 1)], buf.at[(r + 1) % 2], sems[(r + 1) % 2]
                )
            )
        copies[r].wait()
        row = buf.at[r % 2]

        def body(j, carry):
            vmax, vstep = carry
            v = row[pl.ds(j * L, L)]
            m = v > vmax
            vmax = jnp.where(m, v, vmax)
            vstep = jnp.where(m, j, vstep)
            return vmax, vstep

        init = (
            jnp.full((L,), -jnp.inf, jnp.float32),
            jnp.zeros((L,), jnp.int32),
        )
        vmax, vstep = lax.fori_loop(0, STEPS, body, init, unroll=8)
        vidx = vstep * L + iota

        gmax = vmax
        for shift in (1, 2, 4, 8):
            perm = iota ^ shift
            gmax = jnp.maximum(gmax, gmax.at[perm].get(mode="promise_in_bounds"))
        cand = jnp.where(vmax == gmax, vidx, COLS)
        for shift in (1, 2, 4, 8):
            perm = iota ^ shift
            cand = jnp.minimum(cand, cand.at[perm].get(mode="promise_in_bounds"))
        ansvec = jnp.where(iota == r, cand, ansvec)

    res[...] = ansvec
    pltpu.sync_copy(res, out_hbm.at[wid])


def kernel(x):
    out2d = _argmax_sc(x)
    return out2d[:, :RPW].reshape(ROWS)


# speedup vs baseline: 2.7986x; 1.8599x over previous
import jax
import jax.numpy as jnp
from jax import lax
from jax.experimental import pallas as pl
from jax.experimental.pallas import tpu as pltpu

ROWS, COLS = 128, 32768
BR = 8


def _tc_body(x_ref, o_ref):
    xb = x_ref[...]
    m = jnp.max(xb, axis=1, keepdims=True)
    iota = lax.broadcasted_iota(jnp.int32, (BR, COLS), 1)
    idx = jnp.where(xb == m, iota, COLS)
    o_ref[0, 0, :] = jnp.min(idx, axis=1)


def _argmax_tc(x):
    nb = ROWS // BR
    out = pl.pallas_call(
        _tc_body,
        grid=(nb,),
        in_specs=[pl.BlockSpec((BR, COLS), lambda i: (i, 0))],
        out_specs=pl.BlockSpec((1, 1, BR), lambda i: (i, 0, 0)),
        out_shape=jax.ShapeDtypeStruct((nb, 1, BR), jnp.int32),
    )(x)
    return out.reshape(ROWS)


def kernel(x):
    return _argmax_tc(x)


# speedup vs baseline: 3.9934x; 1.4269x over previous
import jax
import jax.numpy as jnp
from jax import lax
from jax.experimental import pallas as pl
from jax.experimental.pallas import tpu as pltpu

ROWS, COLS = 128, 32768
BR = 16


def _tc_body(x_ref, o_ref):
    xb = x_ref[...]
    m = jnp.max(xb, axis=1, keepdims=True)
    iota = lax.broadcasted_iota(jnp.int32, (BR, COLS), 1)
    idx = jnp.where(xb == m, iota, COLS)
    o_ref[0, 0, :] = jnp.min(idx, axis=1)


def _argmax_tc(x):
    nb = ROWS // BR
    out = pl.pallas_call(
        _tc_body,
        grid=(nb,),
        in_specs=[pl.BlockSpec((BR, COLS), lambda i: (i, 0))],
        out_specs=pl.BlockSpec((1, 1, BR), lambda i: (i, 0, 0)),
        out_shape=jax.ShapeDtypeStruct((nb, 1, BR), jnp.int32),
    )(x)
    return out.reshape(ROWS)


def kernel(x):
    return _argmax_tc(x)


# speedup vs baseline: 4.7875x; 1.1989x over previous
import jax
import jax.numpy as jnp
from jax import lax
from jax.experimental import pallas as pl
from jax.experimental.pallas import tpu as pltpu

ROWS, COLS = 128, 32768
BR = 32


def _tc_body(x_ref, o_ref):
    xb = x_ref[...]
    m = jnp.max(xb, axis=1, keepdims=True)
    iota = lax.broadcasted_iota(jnp.int32, (BR, COLS), 1)
    idx = jnp.where(xb == m, iota, COLS)
    o_ref[0, 0, :] = jnp.min(idx, axis=1)


def _argmax_tc(x):
    nb = ROWS // BR
    out = pl.pallas_call(
        _tc_body,
        grid=(nb,),
        in_specs=[pl.BlockSpec((BR, COLS), lambda i: (i, 0))],
        out_specs=pl.BlockSpec((1, 1, BR), lambda i: (i, 0, 0)),
        out_shape=jax.ShapeDtypeStruct((nb, 1, BR), jnp.int32),
    )(x)
    return out.reshape(ROWS)


def kernel(x):
    return _argmax_tc(x)


# speedup vs baseline: 5.0951x; 1.0642x over previous
import jax
import jax.numpy as jnp
from jax import lax
from jax.experimental import pallas as pl
from jax.experimental.pallas import tpu as pltpu

ROWS, COLS = 128, 32768
BR = 64


def _tc_body(x_ref, o_ref):
    xb = x_ref[...]
    m = jnp.max(xb, axis=1, keepdims=True)
    iota = lax.broadcasted_iota(jnp.int32, (BR, COLS), 1)
    idx = jnp.where(xb == m, iota, COLS)
    o_ref[0, 0, :] = jnp.min(idx, axis=1)


def _argmax_tc(x):
    nb = ROWS // BR
    out = pl.pallas_call(
        _tc_body,
        grid=(nb,),
        in_specs=[pl.BlockSpec((BR, COLS), lambda i: (i, 0))],
        out_specs=pl.BlockSpec((1, 1, BR), lambda i: (i, 0, 0)),
        out_shape=jax.ShapeDtypeStruct((nb, 1, BR), jnp.int32),
    )(x)
    return out.reshape(ROWS)


def kernel(x):
    return _argmax_tc(x)
